# baseline (speedup 1.0000x reference)
import jax
import jax.numpy as jnp
from jax import lax
from jax.experimental import pallas as pl
from jax.experimental.pallas import tpu as pltpu
from jax.experimental.pallas import tpu_sc as plsc

N = 8192
D = 768
DO = 768
E = 64
CAP = 320

NC = 2
NS = 16
NW = NC * NS

TB = 256
NB = N // TB
TRASH = E * CAP
XE_ROWS = (E + 1) * CAP

PCK = D // 2
PCKO = DO // 2
TW = N // NW
CH = 64
NCH = TW // CH
CC = 32
NCC = TW // CC
DCH = DO // 16


def _gating_body(x_ref, wg_ref, bg_ref,
                 s1_ref, s2_ref, g1_ref, g2_ref, xb_ref,
                 carry_ref, tri_ref, wgb_ref):
    b = pl.program_id(0)

    @pl.when(b == 0)
    def _():
        carry_ref[...] = jnp.zeros_like(carry_ref)
        ii = lax.broadcasted_iota(jnp.int32, (TB, TB), 0)
        jj = lax.broadcasted_iota(jnp.int32, (TB, TB), 1)
        tri_ref[...] = (jj < ii).astype(jnp.float32)
        wgb_ref[...] = wg_ref[...].astype(jnp.bfloat16)

    xb = x_ref[...].astype(jnp.bfloat16)
    u1 = lax.bitcast_convert_type(xb[:, :PCK], jnp.uint16).astype(jnp.uint32)
    u2 = lax.bitcast_convert_type(xb[:, PCK:], jnp.uint16).astype(jnp.uint32)
    xb_ref[...] = lax.bitcast_convert_type(u1 | (u2 << 16), jnp.int32)
    logits = jnp.dot(xb, wgb_ref[...],
                     preferred_element_type=jnp.float32) + bg_ref[...]

    iota_e = lax.broadcasted_iota(jnp.int32, (TB, E), 1)
    m1 = jnp.max(logits, axis=1, keepdims=True)
    a1 = jnp.min(jnp.where(logits == m1, iota_e, E), axis=1, keepdims=True)
    oh1 = iota_e == a1
    masked = jnp.where(oh1, -jnp.inf, logits)
    m2 = jnp.max(masked, axis=1, keepdims=True)
    a2 = jnp.min(jnp.where(masked == m2, iota_e, E), axis=1, keepdims=True)
    oh2 = iota_e == a2

    t = jnp.exp(m2 - m1)
    den = 1.0 + t
    g1 = 1.0 / den
    g2 = t / den

    ohsum = oh1.astype(jnp.float32) + oh2.astype(jnp.float32)
    S = jnp.dot(tri_ref[...], ohsum,
                preferred_element_type=jnp.float32) + carry_ref[...]
    carry_ref[...] = carry_ref[...] + jnp.sum(ohsum, axis=0, keepdims=True)

    pos1 = jnp.sum(jnp.where(oh1, S, 0.0), axis=1, keepdims=True).astype(jnp.int32)
    pos2 = jnp.sum(jnp.where(oh2, S, 0.0), axis=1, keepdims=True).astype(jnp.int32)
    v1 = pos1 < CAP
    v2 = pos2 < CAP
    s1_ref[...] = jnp.where(v1, a1 * CAP + pos1, TRASH)
    s2_ref[...] = jnp.where(v2, a2 * CAP + pos2, TRASH)
    g1_ref[...] = jnp.broadcast_to(jnp.where(v1, g1, 0.0), (TB, 16))
    g2_ref[...] = jnp.broadcast_to(jnp.where(v2, g2, 0.0), (TB, 16))


def _gating(x, wg, bg2):
    col_i = jax.ShapeDtypeStruct((N, 1), jnp.int32)
    lane_f = jax.ShapeDtypeStruct((N, 16), jnp.float32)
    row_pk = jax.ShapeDtypeStruct((N, PCK), jnp.int32)
    colspec = pl.BlockSpec((TB, 1), lambda b: (b, 0))
    lanespec = pl.BlockSpec((TB, 16), lambda b: (b, 0))
    rowspec = pl.BlockSpec((TB, D), lambda b: (b, 0))
    pkspec = pl.BlockSpec((TB, PCK), lambda b: (b, 0))
    return pl.pallas_call(
        _gating_body,
        grid=(NB,),
        in_specs=[
            rowspec,
            pl.BlockSpec((D, E), lambda b: (0, 0)),
            pl.BlockSpec((1, E), lambda b: (0, 0)),
        ],
        out_specs=[colspec, colspec, lanespec, lanespec, pkspec],
        out_shape=[col_i, col_i, lane_f, lane_f, row_pk],
        scratch_shapes=[pltpu.VMEM((1, E), jnp.float32),
                        pltpu.VMEM((TB, TB), jnp.float32),
                        pltpu.VMEM((D, E), jnp.bfloat16)],
    )(x, wg, bg2)


def _sc_scatter_body(x_hbm, s1_hbm, s2_hbm, xe_hbm,
                     xv0, xv1, i10, i11, i20, i21, lsem0, lsem1, ssem):
    w = lax.axis_index("s") * NC + lax.axis_index("c")
    xv = (xv0, xv1)
    i1 = (i10, i11)
    i2 = (i20, i21)
    lsem = (lsem0, lsem1)

    def start_load(j, b):
        base = w * TW + j * CH
        cps = (pltpu.async_copy(x_hbm.at[pl.ds(base, CH)], xv[b], lsem[b]),
               pltpu.async_copy(s1_hbm.at[pl.ds(base, CH)], i1[b], lsem[b]),
               pltpu.async_copy(s2_hbm.at[pl.ds(base, CH)], i2[b], lsem[b]))
        return cps

    cps = start_load(0, 0)
    for j in range(NCH):
        b = j % 2
        for cp in cps:
            cp.wait()
        if j + 1 < NCH:
            cps = start_load(j + 1, (j + 1) % 2)
        c1 = pltpu.async_copy(xv[b], xe_hbm.at[i1[b]], ssem)
        c2 = pltpu.async_copy(xv[b], xe_hbm.at[i2[b]], ssem)
        c1.wait()
        c2.wait()


_SC_MESH = dict(core_axis_name="c", subcore_axis_name="s",
                num_cores=NC, num_subcores=NS)


def _make_sc_scatter():
    return pl.kernel(
        _sc_scatter_body,
        out_type=jax.ShapeDtypeStruct((XE_ROWS, PCK), jnp.int32),
        mesh=plsc.VectorSubcoreMesh(**_SC_MESH),
        scratch_types=[
            pltpu.VMEM((CH, PCK), jnp.int32),
            pltpu.VMEM((CH, PCK), jnp.int32),
            pltpu.VMEM((CH,), jnp.int32),
            pltpu.VMEM((CH,), jnp.int32),
            pltpu.VMEM((CH,), jnp.int32),
            pltpu.VMEM((CH,), jnp.int32),
            pltpu.SemaphoreType.DMA,
            pltpu.SemaphoreType.DMA,
            pltpu.SemaphoreType.DMA,
        ],
    )


def _ffn_body(xe_ref, we_ref, be_ref, y_ref):
    e = pl.program_id(0)
    xe = xe_ref[...]
    x1 = lax.bitcast_convert_type(xe << 16, jnp.float32).astype(jnp.bfloat16)
    x2 = lax.bitcast_convert_type((xe >> 16) << 16,
                                  jnp.float32).astype(jnp.bfloat16)
    wb = we_ref[0].astype(jnp.bfloat16)
    y = (jnp.dot(x1, wb[:PCK], preferred_element_type=jnp.float32)
         + jnp.dot(x2, wb[PCK:], preferred_element_type=jnp.float32)
         + be_ref[0])
    y = jnp.where(e >= E, 0.0, y)
    yb = y.astype(jnp.bfloat16)
    u1 = lax.bitcast_convert_type(yb[:, :PCKO], jnp.uint16).astype(jnp.uint32)
    u2 = lax.bitcast_convert_type(yb[:, PCKO:], jnp.uint16).astype(jnp.uint32)
    y_ref[...] = lax.bitcast_convert_type(u1 | (u2 << 16), jnp.int32)


def _ffn(xe, we, be):
    return pl.pallas_call(
        _ffn_body,
        grid=(E + 1,),
        in_specs=[
            pl.BlockSpec((CAP, PCK), lambda e: (e, 0)),
            pl.BlockSpec((1, D, DO), lambda e: (jnp.minimum(e, E - 1), 0, 0)),
            pl.BlockSpec((1, 1, DO), lambda e: (jnp.minimum(e, E - 1), 0, 0)),
        ],
        out_specs=pl.BlockSpec((CAP, PCKO), lambda e: (e, 0)),
        out_shape=jax.ShapeDtypeStruct((XE_ROWS, PCKO), jnp.int32),
    )(xe, we, be.reshape(E, 1, DO))


def _sc_combine_body(y_hbm, s1_hbm, s2_hbm, g1_hbm, g2_hbm, o_hbm,
                     ya0, ya1, yb0, yb1, ov, i10, i11, i20, i21,
                     gv10, gv11, gv20, gv21, gsem0, gsem1):
    w = lax.axis_index("s") * NC + lax.axis_index("c")
    ya = (ya0, ya1)
    yb = (yb0, yb1)
    i1 = (i10, i11)
    i2 = (i20, i21)
    gv1 = (gv10, gv11)
    gv2 = (gv20, gv21)
    gsem = (gsem0, gsem1)

    def start_chunk(j, b):
        base = w * TW + j * CC
        pltpu.sync_copy(s1_hbm.at[pl.ds(base, CC)], i1[b])
        pltpu.sync_copy(s2_hbm.at[pl.ds(base, CC)], i2[b])
        return (pltpu.async_copy(y_hbm.at[i1[b]], ya[b], gsem[b]),
                pltpu.async_copy(y_hbm.at[i2[b]], yb[b], gsem[b]),
                pltpu.async_copy(g1_hbm.at[pl.ds(base, CC)], gv1[b], gsem[b]),
                pltpu.async_copy(g2_hbm.at[pl.ds(base, CC)], gv2[b], gsem[b]))

    cps = start_chunk(0, 0)
    for j in range(NCC):
        b = j % 2
        for cp in cps:
            cp.wait()
        if j + 1 < NCC:
            cps = start_chunk(j + 1, (j + 1) % 2)

        yab, ybb, g1b, g2b = ya[b], yb[b], gv1[b], gv2[b]

        def _blend_row(t, _):
            ga = g1b[t]
            gb = g2b[t]
            for c in range(PCKO // 16):
                sl = pl.ds(c * 16, 16)
                w1 = yab[t, sl]
                w2 = ybb[t, sl]
                lo1 = lax.bitcast_convert_type(w1 << 16, jnp.float32)
                lo2 = lax.bitcast_convert_type(w2 << 16, jnp.float32)
                hi1 = lax.bitcast_convert_type((w1 >> 16) << 16, jnp.float32)
                hi2 = lax.bitcast_convert_type((w2 >> 16) << 16, jnp.float32)
                ov[t, sl] = ga * lo1 + gb * lo2
                ov[t, pl.ds(PCKO + c * 16, 16)] = ga * hi1 + gb * hi2
            return ()

        lax.fori_loop(0, CC, _blend_row, ())
        base = w * TW + j * CC
        pltpu.sync_copy(ov, o_hbm.at[pl.ds(base, CC)])


def _make_sc_combine():
    return pl.kernel(
        _sc_combine_body,
        out_type=jax.ShapeDtypeStruct((N, DO), jnp.float32),
        mesh=plsc.VectorSubcoreMesh(**_SC_MESH),
        scratch_types=[
            pltpu.VMEM((CC, PCKO), jnp.int32),
            pltpu.VMEM((CC, PCKO), jnp.int32),
            pltpu.VMEM((CC, PCKO), jnp.int32),
            pltpu.VMEM((CC, PCKO), jnp.int32),
            pltpu.VMEM((CC, DO), jnp.float32),
            pltpu.VMEM((CC,), jnp.int32),
            pltpu.VMEM((CC,), jnp.int32),
            pltpu.VMEM((CC,), jnp.int32),
            pltpu.VMEM((CC,), jnp.int32),
            pltpu.VMEM((CC, 16), jnp.float32),
            pltpu.VMEM((CC, 16), jnp.float32),
            pltpu.VMEM((CC, 16), jnp.float32),
            pltpu.VMEM((CC, 16), jnp.float32),
            pltpu.SemaphoreType.DMA,
            pltpu.SemaphoreType.DMA,
        ],
    )


def kernel(inputs, Wg, bg, We, be):
    bg2 = bg.reshape(1, E)
    s1, s2, g1, g2, xb = _gating(inputs, Wg, bg2)
    s1 = s1.reshape(N)
    s2 = s2.reshape(N)
    xe = _make_sc_scatter()(xb, s1, s2)
    yw = _ffn(xe, We, be)
    return _make_sc_combine()(yw, s1, s2, g1, g2)


# speedup vs baseline: 1.0859x; 1.0859x over previous
import jax
import jax.numpy as jnp
from jax import lax
from jax.experimental import pallas as pl
from jax.experimental.pallas import tpu as pltpu
from jax.experimental.pallas import tpu_sc as plsc

N = 8192
D = 768
DO = 768
E = 64
CAP = 320

NC = 2
NS = 16
NW = NC * NS

TB = 256
NB = N // TB
TRASH = E * CAP
XE_ROWS = (E + 1) * CAP

PCK = D // 2
PCKO = DO // 2
TW = N // NW
CH = 64
NCH = TW // CH
CC = 32
NCC = TW // CC
DCH = DO // 16


def _gating_body(x_ref, wg_ref, bg_ref,
                 s1_ref, s2_ref, g1_ref, g2_ref, xb_ref,
                 carry_ref, tri_ref, wgb_ref):
    b = pl.program_id(0)

    @pl.when(b == 0)
    def _():
        carry_ref[...] = jnp.zeros_like(carry_ref)
        ii = lax.broadcasted_iota(jnp.int32, (TB, TB), 0)
        jj = lax.broadcasted_iota(jnp.int32, (TB, TB), 1)
        tri_ref[...] = (jj < ii).astype(jnp.float32)
        wgb_ref[...] = wg_ref[...].astype(jnp.bfloat16)

    xb = x_ref[...].astype(jnp.bfloat16)
    u1 = lax.bitcast_convert_type(xb[:, :PCK], jnp.uint16).astype(jnp.uint32)
    u2 = lax.bitcast_convert_type(xb[:, PCK:], jnp.uint16).astype(jnp.uint32)
    xb_ref[...] = lax.bitcast_convert_type(u1 | (u2 << 16), jnp.int32)
    logits = jnp.dot(xb, wgb_ref[...],
                     preferred_element_type=jnp.float32) + bg_ref[...]

    iota_e = lax.broadcasted_iota(jnp.int32, (TB, E), 1)
    m1 = jnp.max(logits, axis=1, keepdims=True)
    a1 = jnp.min(jnp.where(logits == m1, iota_e, E), axis=1, keepdims=True)
    oh1 = iota_e == a1
    masked = jnp.where(oh1, -jnp.inf, logits)
    m2 = jnp.max(masked, axis=1, keepdims=True)
    a2 = jnp.min(jnp.where(masked == m2, iota_e, E), axis=1, keepdims=True)
    oh2 = iota_e == a2

    t = jnp.exp(m2 - m1)
    den = 1.0 + t
    g1 = 1.0 / den
    g2 = t / den

    ohsum = oh1.astype(jnp.float32) + oh2.astype(jnp.float32)
    S = jnp.dot(tri_ref[...], ohsum,
                preferred_element_type=jnp.float32) + carry_ref[...]
    carry_ref[...] = carry_ref[...] + jnp.sum(ohsum, axis=0, keepdims=True)

    pos1 = jnp.sum(jnp.where(oh1, S, 0.0), axis=1, keepdims=True).astype(jnp.int32)
    pos2 = jnp.sum(jnp.where(oh2, S, 0.0), axis=1, keepdims=True).astype(jnp.int32)
    v1 = pos1 < CAP
    v2 = pos2 < CAP
    s1_ref[...] = jnp.where(v1, a1 * CAP + pos1, TRASH)
    s2_ref[...] = jnp.where(v2, a2 * CAP + pos2, TRASH)
    g1_ref[...] = jnp.broadcast_to(jnp.where(v1, g1, 0.0), (TB, 16))
    g2_ref[...] = jnp.broadcast_to(jnp.where(v2, g2, 0.0), (TB, 16))


def _gating(x, wg, bg2):
    col_i = jax.ShapeDtypeStruct((N, 1), jnp.int32)
    lane_f = jax.ShapeDtypeStruct((N, 16), jnp.float32)
    row_pk = jax.ShapeDtypeStruct((N, PCK), jnp.int32)
    colspec = pl.BlockSpec((TB, 1), lambda b: (b, 0))
    lanespec = pl.BlockSpec((TB, 16), lambda b: (b, 0))
    rowspec = pl.BlockSpec((TB, D), lambda b: (b, 0))
    pkspec = pl.BlockSpec((TB, PCK), lambda b: (b, 0))
    return pl.pallas_call(
        _gating_body,
        grid=(NB,),
        in_specs=[
            rowspec,
            pl.BlockSpec((D, E), lambda b: (0, 0)),
            pl.BlockSpec((1, E), lambda b: (0, 0)),
        ],
        out_specs=[colspec, colspec, lanespec, lanespec, pkspec],
        out_shape=[col_i, col_i, lane_f, lane_f, row_pk],
        scratch_shapes=[pltpu.VMEM((1, E), jnp.float32),
                        pltpu.VMEM((TB, TB), jnp.float32),
                        pltpu.VMEM((D, E), jnp.bfloat16)],
    )(x, wg, bg2)


def _sc_scatter_body(x_hbm, s1_hbm, s2_hbm, xe_hbm,
                     xv0, xv1, i10, i11, i20, i21, lsem0, lsem1, ssem):
    w = lax.axis_index("s") * NC + lax.axis_index("c")
    xv = (xv0, xv1)
    i1 = (i10, i11)
    i2 = (i20, i21)
    lsem = (lsem0, lsem1)

    def start_load(j, b):
        base = w * TW + j * CH
        cps = (pltpu.async_copy(x_hbm.at[pl.ds(base, CH)], xv[b], lsem[b]),
               pltpu.async_copy(s1_hbm.at[pl.ds(base, CH)], i1[b], lsem[b]),
               pltpu.async_copy(s2_hbm.at[pl.ds(base, CH)], i2[b], lsem[b]))
        return cps

    cps = start_load(0, 0)
    for j in range(NCH):
        b = j % 2
        for cp in cps:
            cp.wait()
        if j + 1 < NCH:
            cps = start_load(j + 1, (j + 1) % 2)
        c1 = pltpu.async_copy(xv[b], xe_hbm.at[i1[b]], ssem)
        c2 = pltpu.async_copy(xv[b], xe_hbm.at[i2[b]], ssem)
        c1.wait()
        c2.wait()


_SC_MESH = dict(core_axis_name="c", subcore_axis_name="s",
                num_cores=NC, num_subcores=NS)


def _make_sc_scatter():
    return pl.kernel(
        _sc_scatter_body,
        out_type=jax.ShapeDtypeStruct((XE_ROWS, PCK), jnp.int32),
        mesh=plsc.VectorSubcoreMesh(**_SC_MESH),
        scratch_types=[
            pltpu.VMEM((CH, PCK), jnp.int32),
            pltpu.VMEM((CH, PCK), jnp.int32),
            pltpu.VMEM((CH,), jnp.int32),
            pltpu.VMEM((CH,), jnp.int32),
            pltpu.VMEM((CH,), jnp.int32),
            pltpu.VMEM((CH,), jnp.int32),
            pltpu.SemaphoreType.DMA,
            pltpu.SemaphoreType.DMA,
            pltpu.SemaphoreType.DMA,
        ],
    )


def _ffn_body(xe_ref, we_ref, be_ref, y_ref):
    e = pl.program_id(0)
    xe = xe_ref[...]
    x1 = lax.bitcast_convert_type(xe << 16, jnp.float32).astype(jnp.bfloat16)
    x2 = lax.bitcast_convert_type((xe >> 16) << 16,
                                  jnp.float32).astype(jnp.bfloat16)
    wb = we_ref[0].astype(jnp.bfloat16)
    y = (jnp.dot(x1, wb[:PCK], preferred_element_type=jnp.float32)
         + jnp.dot(x2, wb[PCK:], preferred_element_type=jnp.float32)
         + be_ref[0])
    y_ref[...] = jnp.where(e >= E, 0.0, y)


def _ffn(xe, we, be):
    return pl.pallas_call(
        _ffn_body,
        grid=(E + 1,),
        in_specs=[
            pl.BlockSpec((CAP, PCK), lambda e: (e, 0)),
            pl.BlockSpec((1, D, DO), lambda e: (jnp.minimum(e, E - 1), 0, 0)),
            pl.BlockSpec((1, 1, DO), lambda e: (jnp.minimum(e, E - 1), 0, 0)),
        ],
        out_specs=pl.BlockSpec((CAP, DO), lambda e: (e, 0)),
        out_shape=jax.ShapeDtypeStruct((XE_ROWS, DO), jnp.float32),
    )(xe, we, be.reshape(E, 1, DO))


def _sc_combine_body(y_hbm, s1_hbm, s2_hbm, g1_hbm, g2_hbm, o_hbm,
                     ya0, ya1, yb0, yb1, i10, i11, i20, i21,
                     gv10, gv11, gv20, gv21, gsem0, gsem1):
    w = lax.axis_index("s") * NC + lax.axis_index("c")
    ya = (ya0, ya1)
    yb = (yb0, yb1)
    i1 = (i10, i11)
    i2 = (i20, i21)
    gv1 = (gv10, gv11)
    gv2 = (gv20, gv21)
    gsem = (gsem0, gsem1)

    def start_chunk(j, b):
        base = w * TW + j * CC
        pltpu.sync_copy(s1_hbm.at[pl.ds(base, CC)], i1[b])
        pltpu.sync_copy(s2_hbm.at[pl.ds(base, CC)], i2[b])
        return (pltpu.async_copy(y_hbm.at[i1[b]], ya[b], gsem[b]),
                pltpu.async_copy(y_hbm.at[i2[b]], yb[b], gsem[b]),
                pltpu.async_copy(g1_hbm.at[pl.ds(base, CC)], gv1[b], gsem[b]),
                pltpu.async_copy(g2_hbm.at[pl.ds(base, CC)], gv2[b], gsem[b]))

    cps = start_chunk(0, 0)
    for j in range(NCC):
        b = j % 2
        for cp in cps:
            cp.wait()
        if j + 1 < NCC:
            cps = start_chunk(j + 1, (j + 1) % 2)

        yab, ybb, g1b, g2b = ya[b], yb[b], gv1[b], gv2[b]

        def _blend_row(t, _):
            ga = g1b[t]
            gb = g2b[t]
            for c in range(DCH):
                sl = pl.ds(c * 16, 16)
                yab[t, sl] = ga * yab[t, sl] + gb * ybb[t, sl]
            return ()

        lax.fori_loop(0, CC, _blend_row, ())
        base = w * TW + j * CC
        pltpu.sync_copy(yab, o_hbm.at[pl.ds(base, CC)])


def _make_sc_combine():
    return pl.kernel(
        _sc_combine_body,
        out_type=jax.ShapeDtypeStruct((N, DO), jnp.float32),
        mesh=plsc.VectorSubcoreMesh(**_SC_MESH),
        scratch_types=[
            pltpu.VMEM((CC, DO), jnp.float32),
            pltpu.VMEM((CC, DO), jnp.float32),
            pltpu.VMEM((CC, DO), jnp.float32),
            pltpu.VMEM((CC, DO), jnp.float32),
            pltpu.VMEM((CC,), jnp.int32),
            pltpu.VMEM((CC,), jnp.int32),
            pltpu.VMEM((CC,), jnp.int32),
            pltpu.VMEM((CC,), jnp.int32),
            pltpu.VMEM((CC, 16), jnp.float32),
            pltpu.VMEM((CC, 16), jnp.float32),
            pltpu.VMEM((CC, 16), jnp.float32),
            pltpu.VMEM((CC, 16), jnp.float32),
            pltpu.SemaphoreType.DMA,
            pltpu.SemaphoreType.DMA,
        ],
    )


def kernel(inputs, Wg, bg, We, be):
    bg2 = bg.reshape(1, E)
    s1, s2, g1, g2, xb = _gating(inputs, Wg, bg2)
    s1 = s1.reshape(N)
    s2 = s2.reshape(N)
    xe = _make_sc_scatter()(xb, s1, s2)
    yw = _ffn(xe, We, be)
    return _make_sc_combine()(yw, s1, s2, g1, g2)


# speedup vs baseline: 1.1103x; 1.0225x over previous
import jax
import jax.numpy as jnp
from jax import lax
from jax.experimental import pallas as pl
from jax.experimental.pallas import tpu as pltpu
from jax.experimental.pallas import tpu_sc as plsc

N = 8192
D = 768
DO = 768
E = 64
CAP = 320

NC = 2
NS = 16
NW = NC * NS

TB = 256
NB = N // TB
TRASH = E * CAP
XE_ROWS = (E + 1) * CAP

PCK = D // 2
PCKO = DO // 2
TW = N // NW
CH = 128
NCH = TW // CH
CC = 32
NCC = TW // CC
DCH = DO // 16


def _gating_body(x_ref, wg_ref, bg_ref,
                 s1_ref, s2_ref, g1_ref, g2_ref, xb_ref,
                 carry_ref, tri_ref, wgb_ref):
    b = pl.program_id(0)

    @pl.when(b == 0)
    def _():
        carry_ref[...] = jnp.zeros_like(carry_ref)
        ii = lax.broadcasted_iota(jnp.int32, (TB, TB), 0)
        jj = lax.broadcasted_iota(jnp.int32, (TB, TB), 1)
        tri_ref[...] = (jj < ii).astype(jnp.float32)
        wgb_ref[...] = wg_ref[...].astype(jnp.bfloat16)

    xb = x_ref[...].astype(jnp.bfloat16)
    u1 = lax.bitcast_convert_type(xb[:, :PCK], jnp.uint16).astype(jnp.uint32)
    u2 = lax.bitcast_convert_type(xb[:, PCK:], jnp.uint16).astype(jnp.uint32)
    xb_ref[...] = lax.bitcast_convert_type(u1 | (u2 << 16), jnp.int32)
    logits = jnp.dot(xb, wgb_ref[...],
                     preferred_element_type=jnp.float32) + bg_ref[...]

    iota_e = lax.broadcasted_iota(jnp.int32, (TB, E), 1)
    m1 = jnp.max(logits, axis=1, keepdims=True)
    a1 = jnp.min(jnp.where(logits == m1, iota_e, E), axis=1, keepdims=True)
    oh1 = iota_e == a1
    masked = jnp.where(oh1, -jnp.inf, logits)
    m2 = jnp.max(masked, axis=1, keepdims=True)
    a2 = jnp.min(jnp.where(masked == m2, iota_e, E), axis=1, keepdims=True)
    oh2 = iota_e == a2

    t = jnp.exp(m2 - m1)
    den = 1.0 + t
    g1 = 1.0 / den
    g2 = t / den

    ohsum = oh1.astype(jnp.float32) + oh2.astype(jnp.float32)
    S = jnp.dot(tri_ref[...], ohsum,
                preferred_element_type=jnp.float32) + carry_ref[...]
    carry_ref[...] = carry_ref[...] + jnp.sum(ohsum, axis=0, keepdims=True)

    pos1 = jnp.sum(jnp.where(oh1, S, 0.0), axis=1, keepdims=True).astype(jnp.int32)
    pos2 = jnp.sum(jnp.where(oh2, S, 0.0), axis=1, keepdims=True).astype(jnp.int32)
    v1 = pos1 < CAP
    v2 = pos2 < CAP
    s1_ref[...] = jnp.where(v1, a1 * CAP + pos1, TRASH)
    s2_ref[...] = jnp.where(v2, a2 * CAP + pos2, TRASH)
    g1_ref[...] = jnp.broadcast_to(jnp.where(v1, g1, 0.0), (TB, 16))
    g2_ref[...] = jnp.broadcast_to(jnp.where(v2, g2, 0.0), (TB, 16))


def _gating(x, wg, bg2):
    col_i = jax.ShapeDtypeStruct((N, 1), jnp.int32)
    lane_f = jax.ShapeDtypeStruct((N, 16), jnp.float32)
    row_pk = jax.ShapeDtypeStruct((N, PCK), jnp.int32)
    colspec = pl.BlockSpec((TB, 1), lambda b: (b, 0))
    lanespec = pl.BlockSpec((TB, 16), lambda b: (b, 0))
    rowspec = pl.BlockSpec((TB, D), lambda b: (b, 0))
    pkspec = pl.BlockSpec((TB, PCK), lambda b: (b, 0))
    return pl.pallas_call(
        _gating_body,
        grid=(NB,),
        in_specs=[
            rowspec,
            pl.BlockSpec((D, E), lambda b: (0, 0)),
            pl.BlockSpec((1, E), lambda b: (0, 0)),
        ],
        out_specs=[colspec, colspec, lanespec, lanespec, pkspec],
        out_shape=[col_i, col_i, lane_f, lane_f, row_pk],
        scratch_shapes=[pltpu.VMEM((1, E), jnp.float32),
                        pltpu.VMEM((TB, TB), jnp.float32),
                        pltpu.VMEM((D, E), jnp.bfloat16)],
    )(x, wg, bg2)


def _sc_scatter_body(x_hbm, s1_hbm, s2_hbm, xe_hbm,
                     xv0, xv1, i10, i11, i20, i21, lsem0, lsem1, ssem):
    w = lax.axis_index("s") * NC + lax.axis_index("c")
    xv = (xv0, xv1)
    i1 = (i10, i11)
    i2 = (i20, i21)
    lsem = (lsem0, lsem1)

    def start_load(j, b):
        base = w * TW + j * CH
        cps = (pltpu.async_copy(x_hbm.at[pl.ds(base, CH)], xv[b], lsem[b]),
               pltpu.async_copy(s1_hbm.at[pl.ds(base, CH)], i1[b], lsem[b]),
               pltpu.async_copy(s2_hbm.at[pl.ds(base, CH)], i2[b], lsem[b]))
        return cps

    cps = start_load(0, 0)
    for j in range(NCH):
        b = j % 2
        for cp in cps:
            cp.wait()
        if j + 1 < NCH:
            cps = start_load(j + 1, (j + 1) % 2)
        c1 = pltpu.async_copy(xv[b], xe_hbm.at[i1[b]], ssem)
        c2 = pltpu.async_copy(xv[b], xe_hbm.at[i2[b]], ssem)
        c1.wait()
        c2.wait()


_SC_MESH = dict(core_axis_name="c", subcore_axis_name="s",
                num_cores=NC, num_subcores=NS)


def _make_sc_scatter():
    return pl.kernel(
        _sc_scatter_body,
        out_type=jax.ShapeDtypeStruct((XE_ROWS, PCK), jnp.int32),
        mesh=plsc.VectorSubcoreMesh(**_SC_MESH),
        scratch_types=[
            pltpu.VMEM((CH, PCK), jnp.int32),
            pltpu.VMEM((CH, PCK), jnp.int32),
            pltpu.VMEM((CH,), jnp.int32),
            pltpu.VMEM((CH,), jnp.int32),
            pltpu.VMEM((CH,), jnp.int32),
            pltpu.VMEM((CH,), jnp.int32),
            pltpu.SemaphoreType.DMA,
            pltpu.SemaphoreType.DMA,
            pltpu.SemaphoreType.DMA,
        ],
    )


def _ffn_body(xe_ref, we_ref, be_ref, y_ref):
    e = pl.program_id(0)
    xe = xe_ref[...]
    x1 = lax.bitcast_convert_type(xe << 16, jnp.float32).astype(jnp.bfloat16)
    x2 = lax.bitcast_convert_type((xe >> 16) << 16,
                                  jnp.float32).astype(jnp.bfloat16)
    wb = we_ref[0].astype(jnp.bfloat16)
    y = (jnp.dot(x1, wb[:PCK], preferred_element_type=jnp.float32)
         + jnp.dot(x2, wb[PCK:], preferred_element_type=jnp.float32)
         + be_ref[0])
    y_ref[...] = jnp.where(e >= E, 0.0, y)


def _ffn(xe, we, be):
    return pl.pallas_call(
        _ffn_body,
        grid=(E + 1,),
        in_specs=[
            pl.BlockSpec((CAP, PCK), lambda e: (e, 0)),
            pl.BlockSpec((1, D, DO), lambda e: (jnp.minimum(e, E - 1), 0, 0)),
            pl.BlockSpec((1, 1, DO), lambda e: (jnp.minimum(e, E - 1), 0, 0)),
        ],
        out_specs=pl.BlockSpec((CAP, DO), lambda e: (e, 0)),
        out_shape=jax.ShapeDtypeStruct((XE_ROWS, DO), jnp.float32),
    )(xe, we, be.reshape(E, 1, DO))


def _sc_combine_body(y_hbm, s1_hbm, s2_hbm, g1_hbm, g2_hbm, o_hbm,
                     ya0, ya1, yb0, yb1, i10, i11, i20, i21,
                     gv10, gv11, gv20, gv21, gsem0, gsem1, isem):
    w = lax.axis_index("s") * NC + lax.axis_index("c")
    ya = (ya0, ya1)
    yb = (yb0, yb1)
    i1 = (i10, i11)
    i2 = (i20, i21)
    gv1 = (gv10, gv11)
    gv2 = (gv20, gv21)
    gsem = (gsem0, gsem1)

    def load_idx(j, b):
        base = w * TW + j * CC
        return (pltpu.async_copy(s1_hbm.at[pl.ds(base, CC)], i1[b], isem),
                pltpu.async_copy(s2_hbm.at[pl.ds(base, CC)], i2[b], isem))

    def start_gathers(j, b):
        base = w * TW + j * CC
        return (pltpu.async_copy(y_hbm.at[i1[b]], ya[b], gsem[b]),
                pltpu.async_copy(y_hbm.at[i2[b]], yb[b], gsem[b]),
                pltpu.async_copy(g1_hbm.at[pl.ds(base, CC)], gv1[b], gsem[b]),
                pltpu.async_copy(g2_hbm.at[pl.ds(base, CC)], gv2[b], gsem[b]))

    for cp in load_idx(0, 0):
        cp.wait()
    cps = start_gathers(0, 0)
    icps = load_idx(1, 1)
    for j in range(NCC):
        b = j % 2
        for cp in cps:
            cp.wait()
        if j + 1 < NCC:
            for cp in icps:
                cp.wait()
            cps = start_gathers(j + 1, (j + 1) % 2)
            if j + 2 < NCC:
                icps = load_idx(j + 2, b)

        yab, ybb, g1b, g2b = ya[b], yb[b], gv1[b], gv2[b]

        def _blend_row(t, _):
            ga = g1b[t]
            gb = g2b[t]
            for c in range(DCH):
                sl = pl.ds(c * 16, 16)
                yab[t, sl] = ga * yab[t, sl] + gb * ybb[t, sl]
            return ()

        lax.fori_loop(0, CC, _blend_row, ())
        base = w * TW + j * CC
        pltpu.sync_copy(yab, o_hbm.at[pl.ds(base, CC)])


def _make_sc_combine():
    return pl.kernel(
        _sc_combine_body,
        out_type=jax.ShapeDtypeStruct((N, DO), jnp.float32),
        mesh=plsc.VectorSubcoreMesh(**_SC_MESH),
        scratch_types=[
            pltpu.VMEM((CC, DO), jnp.float32),
            pltpu.VMEM((CC, DO), jnp.float32),
            pltpu.VMEM((CC, DO), jnp.float32),
            pltpu.VMEM((CC, DO), jnp.float32),
            pltpu.VMEM((CC,), jnp.int32),
            pltpu.VMEM((CC,), jnp.int32),
            pltpu.VMEM((CC,), jnp.int32),
            pltpu.VMEM((CC,), jnp.int32),
            pltpu.VMEM((CC, 16), jnp.float32),
            pltpu.VMEM((CC, 16), jnp.float32),
            pltpu.VMEM((CC, 16), jnp.float32),
            pltpu.VMEM((CC, 16), jnp.float32),
            pltpu.SemaphoreType.DMA,
            pltpu.SemaphoreType.DMA,
            pltpu.SemaphoreType.DMA,
        ],
    )


def kernel(inputs, Wg, bg, We, be):
    bg2 = bg.reshape(1, E)
    s1, s2, g1, g2, xb = _gating(inputs, Wg, bg2)
    s1 = s1.reshape(N)
    s2 = s2.reshape(N)
    xe = _make_sc_scatter()(xb, s1, s2)
    yw = _ffn(xe, We, be)
    return _make_sc_combine()(yw, s1, s2, g1, g2)
